# Initial kernel scaffold; baseline (speedup 1.0000x reference)
import jax
import jax.numpy as jnp
from jax.experimental import pallas as pl


def kernel(features, prototypes, labels):
    raise NotImplementedError("write your pallas kernel here")


# speedup vs baseline: 1.3817x; 1.3817x over previous
import functools

import jax
import jax.numpy as jnp
from jax import lax
from jax.experimental import pallas as pl
from jax.experimental.pallas import tpu as pltpu
from jax.experimental.pallas import tpu_sc as plsc


def _make_sc_partials(N, D, C, NC, NS, L):
    NW = NC * NS
    RW = N // NW
    CH = 64
    NCH = RW // CH
    G = CH // L
    assert N == NW * RW and RW == NCH * CH and CH == G * L and D % L == 0

    mesh = plsc.VectorSubcoreMesh(core_axis_name="c", subcore_axis_name="s")

    @functools.partial(
        pl.kernel,
        mesh=mesh,
        compiler_params=pltpu.CompilerParams(needs_layout_passes=False),
        out_type=jax.ShapeDtypeStruct((NW, L), jnp.float32),
        scratch_types=[
            pltpu.VMEM((CH,), jnp.int32),
            pltpu.VMEM((CH, D), jnp.float32),
            pltpu.VMEM((CH, D), jnp.float32),
            pltpu.VMEM((L,), jnp.float32),
            pltpu.SemaphoreType.DMA,
        ],
    )
    def kern(f_hbm, p_hbm, lab_hbm, out_hbm,
             idx_v, f_v, p_v, accv, sem):
        wid = lax.axis_index("s") * NC + lax.axis_index("c")
        lanes = lax.broadcasted_iota(jnp.int32, (L,), 0)
        zeros = jnp.zeros((L,), jnp.float32)

        def group_body(g, acc):
            rbase = g * L
            sfp = zeros
            sden = zeros
            for r in range(L):
                row = rbase + r
                fp = ff = pp = None
                for j in range(D // L):
                    fv = f_v[row, pl.ds(j * L, L)]
                    pv = p_v[row, pl.ds(j * L, L)]
                    if fp is None:
                        fp, ff, pp = fv * pv, fv * fv, pv * pv
                    else:
                        fp = fp + fv * pv
                        ff = ff + fv * fv
                        pp = pp + pv * pv
                s_fp = jnp.sum(fp)
                s_den = jnp.sum(ff) * jnp.sum(pp)
                m = lanes == r
                sfp = jnp.where(m, s_fp, sfp)
                sden = jnp.where(m, s_den, sden)

            den = jnp.maximum(sden, jnp.float32(1e-24))
            seed = jnp.int32(0x5F3759DF) - lax.shift_right_arithmetic(
                plsc.bitcast(den, jnp.int32), 1)
            y = plsc.bitcast(seed, jnp.float32)
            for _ in range(3):
                y = y * (jnp.float32(1.5) - jnp.float32(0.5) * den * y * y)
            return acc + sfp * y

        def chunk_body(ch, acc):
            base = wid * RW + ch * CH
            pltpu.sync_copy(lab_hbm.at[pl.ds(base, CH)], idx_v)
            pltpu.async_copy(p_hbm.at[idx_v], p_v, sem).wait()
            pltpu.sync_copy(f_hbm.at[pl.ds(base, CH)], f_v)
            return lax.fori_loop(0, G, group_body, acc)

        acc = lax.fori_loop(0, NCH, chunk_body, jnp.zeros((L,), jnp.float32))
        accv[...] = acc
        pltpu.sync_copy(accv, out_hbm.at[wid])

    return kern


def kernel(features, prototypes, labels):
    N, D = features.shape
    C = prototypes.shape[0]
    info = plsc.get_sparse_core_info()
    NC, NS, L = info.num_cores, info.num_subcores, info.num_lanes
    kern = _make_sc_partials(N, D, C, NC, NS, L)
    partials = kern(features.astype(jnp.float32),
                    prototypes.astype(jnp.float32),
                    labels.astype(jnp.int32))
    return -(jnp.sum(partials) / jnp.float32(N))


# speedup vs baseline: 1.5941x; 1.1537x over previous
import functools

import jax
import jax.numpy as jnp
from jax import lax
from jax.experimental import pallas as pl
from jax.experimental.pallas import tpu as pltpu
from jax.experimental.pallas import tpu_sc as plsc


def _make_sc_partials(N, D, C, NC, NS, L):
    NW = NC * NS
    RW = N // NW
    CH = 32
    NCH = RW // CH
    G = CH // L
    assert N == NW * RW and RW == NCH * CH and CH == G * L and D % L == 0
    assert NCH % 2 == 0

    mesh = plsc.VectorSubcoreMesh(core_axis_name="c", subcore_axis_name="s")

    @functools.partial(
        pl.kernel,
        mesh=mesh,
        compiler_params=pltpu.CompilerParams(needs_layout_passes=False),
        out_type=jax.ShapeDtypeStruct((NW, L), jnp.float32),
        scratch_types=[
            pltpu.VMEM((RW,), jnp.int32),
            pltpu.VMEM((CH, D), jnp.float32),
            pltpu.VMEM((CH, D), jnp.float32),
            pltpu.VMEM((CH, D), jnp.float32),
            pltpu.VMEM((CH, D), jnp.float32),
            pltpu.VMEM((L,), jnp.float32),
            pltpu.SemaphoreType.DMA,
            pltpu.SemaphoreType.DMA,
            pltpu.SemaphoreType.DMA,
            pltpu.SemaphoreType.DMA,
        ],
    )
    def kern(f_hbm, p_hbm, lab_hbm, out_hbm,
             idx_v, f0, f1, p0, p1, accv,
             semf0, semf1, semg0, semg1):
        cid = lax.axis_index("c")
        sid = lax.axis_index("s")
        wid = sid * NC + cid
        lanes = lax.broadcasted_iota(jnp.int32, (L,), 0)
        zeros = jnp.zeros((L,), jnp.float32)

        pltpu.sync_copy(lab_hbm.at[pl.ds(wid * RW, RW)], idx_v)

        def start_chunk(ch, f_buf, p_buf, semf, semg):
            base = wid * RW + ch * CH
            cpf = pltpu.async_copy(f_hbm.at[pl.ds(base, CH)], f_buf, semf)
            idx = idx_v.at[pl.ds(ch * CH, CH)]
            cpg = pltpu.async_copy(p_hbm.at[idx], p_buf, semg)
            return cpf, cpg

        def wait_chunk(f_buf, p_buf, semf, semg):
            pltpu.make_async_copy(f_hbm.at[pl.ds(0, CH)], f_buf, semf).wait()
            pltpu.make_async_copy(p_hbm.at[pl.ds(0, CH)], p_buf, semg).wait()

        def compute(f_v, p_v, acc):
            def group_body(g, acc):
                rbase = g * L
                sfp = zeros
                sden = zeros
                for r in range(L):
                    row = rbase + r
                    fp = ff = pp = None
                    for j in range(D // L):
                        fv = f_v[row, pl.ds(j * L, L)]
                        pv = p_v[row, pl.ds(j * L, L)]
                        if fp is None:
                            fp, ff, pp = fv * pv, fv * fv, pv * pv
                        else:
                            fp = fp + fv * pv
                            ff = ff + fv * fv
                            pp = pp + pv * pv
                    s_fp = jnp.sum(fp)
                    s_den = jnp.sum(ff) * jnp.sum(pp)
                    m = lanes == r
                    sfp = jnp.where(m, s_fp, sfp)
                    sden = jnp.where(m, s_den, sden)

                den = jnp.maximum(sden, jnp.float32(1e-24))
                seed = jnp.int32(0x5F3759DF) - lax.shift_right_arithmetic(
                    plsc.bitcast(den, jnp.int32), 1)
                y = plsc.bitcast(seed, jnp.float32)
                for _ in range(3):
                    y = y * (jnp.float32(1.5) - jnp.float32(0.5) * den * y * y)
                return acc + sfp * y

            return lax.fori_loop(0, G, group_body, acc)

        start_chunk(0, f0, p0, semf0, semg0)

        def pair_body(i, acc):
            ch0 = 2 * i
            start_chunk(ch0 + 1, f1, p1, semf1, semg1)
            wait_chunk(f0, p0, semf0, semg0)
            acc = compute(f0, p0, acc)
            start_chunk(jnp.minimum(ch0 + 2, NCH - 2), f0, p0, semf0, semg0)
            wait_chunk(f1, p1, semf1, semg1)
            acc = compute(f1, p1, acc)
            return acc

        acc = lax.fori_loop(0, NCH // 2, pair_body, zeros)
        wait_chunk(f0, p0, semf0, semg0)

        accv[...] = acc
        pltpu.sync_copy(accv, out_hbm.at[wid])

    return kern


def kernel(features, prototypes, labels):
    N, D = features.shape
    C = prototypes.shape[0]
    info = plsc.get_sparse_core_info()
    NC, NS, L = info.num_cores, info.num_subcores, info.num_lanes
    kern = _make_sc_partials(N, D, C, NC, NS, L)
    partials = kern(features.astype(jnp.float32),
                    prototypes.astype(jnp.float32),
                    labels.astype(jnp.int32))
    return -(jnp.sum(partials) / jnp.float32(N))
